# Initial kernel scaffold; baseline (speedup 1.0000x reference)
import jax
import jax.numpy as jnp
from jax.experimental import pallas as pl


def kernel(img, transform):
    raise NotImplementedError("write your pallas kernel here")


# speedup vs baseline: 14.7600x; 14.7600x over previous
import functools
import jax
import jax.numpy as jnp
from jax import lax
from jax.experimental import pallas as pl
from jax.experimental.pallas import tpu as pltpu
from jax.experimental.pallas import tpu_sc as plsc

H = 512
W = 512
NB = 32
OH = 256
OW = 256
D = 8
RCH = 8
P = RCH * OW
K = P // 128
NCHUNK = OH // RCH
ZERO_ROW = H * W


def _body(img_hbm, tr_hbm, out_hbm, t_v, idx_v, g_v, sem):
    wid = lax.axis_index("s") * 2 + lax.axis_index("c")
    b = wid

    pltpu.sync_copy(tr_hbm.at[b], t_v)
    tv = [t_v[j, :] for j in range(9)]
    t22 = tv[8]
    t0, t1, t2, t3, t4, t5, t6, t7 = [tv[j] / t22 for j in range(8)]
    xiota = jnp.arange(16, dtype=jnp.int32).astype(jnp.float32)

    def chunk_body(ci, carry):
        y0 = ci * RCH

        def row_body(r, carry2):
            yf = jnp.full((16,), y0 + r, jnp.int32).astype(jnp.float32)
            ay = t1 * yf
            cy = t4 * yf
            ky = t7 * yf
            for vx in range(16):
                xf = xiota + jnp.float32(vx * 16)
                k = (t6 * xf + ky) + 1.0
                xi = (((t0 * xf + ay) + t2) / k).astype(jnp.int32)
                yi = (((t3 * xf + cy) + t5) / k).astype(jnp.int32)
                valid = (xi >= 0) & (xi < W) & (yi >= 0) & (yi < H)
                idx = jnp.where(valid, yi * W + xi, ZERO_ROW)
                idx_v[2 * r + (vx // 8), pl.ds((vx % 8) * 16, 16)] = idx
            return carry2

        lax.fori_loop(0, RCH, row_body, carry)

        def fire(j, carry2):
            pltpu.make_async_copy(
                img_hbm.at[idx_v.at[j]],
                g_v.at[pl.ds(j * 128, 128)],
                sem).start()
            return carry2

        lax.fori_loop(0, K, fire, carry)

        def drain(j, carry2):
            pltpu.make_async_copy(
                img_hbm.at[idx_v.at[j]],
                g_v.at[pl.ds(j * 128, 128)],
                sem).wait()
            return carry2

        lax.fori_loop(0, K, drain, carry)

        pltpu.sync_copy(g_v.at[:, pl.ds(0, 3)],
                        out_hbm.at[b, pl.ds(ci * P, P)])
        return carry

    lax.fori_loop(0, NCHUNK, chunk_body, 0)


@jax.jit
def _run(img_aug, tr_rep):
    mesh = plsc.VectorSubcoreMesh(core_axis_name="c", subcore_axis_name="s")
    f = pl.kernel(
        _body,
        mesh=mesh,
        compiler_params=pltpu.CompilerParams(use_tc_tiling_on_sc=False),
        out_type=jax.ShapeDtypeStruct((NB, OH * OW, 3), jnp.float32),
        scratch_types=[
            pltpu.VMEM((9, 16), jnp.float32),
            pltpu.VMEM((K, 128), jnp.int32),
            pltpu.VMEM((P, D), jnp.float32),
            pltpu.SemaphoreType.DMA,
        ],
    )
    return f(img_aug, tr_rep).reshape(NB, OH, OW, 3)


def kernel(img, transform):
    img_aug = jnp.pad(img.reshape(H * W, 3), ((0, 1), (0, D - 3)))
    tr_rep = jnp.broadcast_to(transform.reshape(NB, 9, 1), (NB, 9, 16))
    return _run(img_aug, tr_rep)


# speedup vs baseline: 16.3877x; 1.1103x over previous
import functools
import jax
import jax.numpy as jnp
from jax import lax
from jax.experimental import pallas as pl
from jax.experimental.pallas import tpu as pltpu
from jax.experimental.pallas import tpu_sc as plsc

H = 512
W = 512
NB = 32
OH = 256
OW = 256
D = 8
RCH = 8
P = RCH * OW
K = P // 128
NCHUNK = OH // RCH
ZERO_ROW = H * W


def _body(img_hbm, tr_hbm, out_hbm, t_v, idx_v, g_v, sem):
    wid = lax.axis_index("s") * 2 + lax.axis_index("c")
    b = wid

    pltpu.sync_copy(tr_hbm.at[b], t_v)
    tv = [t_v[j, :] for j in range(9)]
    t22 = tv[8]
    t0, t1, t2, t3, t4, t5, t6, t7 = [tv[j] / t22 for j in range(8)]
    xiota = jnp.arange(16, dtype=jnp.int32).astype(jnp.float32)

    def chunk_body(ci, carry):
        y0 = ci * RCH

        def row_body(r, carry2):
            yf = jnp.full((16,), y0 + r, jnp.int32).astype(jnp.float32)
            ay = t1 * yf
            cy = t4 * yf
            ky = t7 * yf
            for vx in range(16):
                xf = xiota + jnp.float32(vx * 16)
                k = (t6 * xf + ky) + 1.0
                xi = (((t0 * xf + ay) + t2) / k).astype(jnp.int32)
                yi = (((t3 * xf + cy) + t5) / k).astype(jnp.int32)
                valid = (xi >= 0) & (xi < W) & (yi >= 0) & (yi < H)
                idx = jnp.where(valid, yi * W + xi, ZERO_ROW)
                idx_v[2 * r + (vx // 8), pl.ds((vx % 8) * 16, 16)] = idx
            return carry2

        lax.fori_loop(0, RCH, row_body, carry)

        def fire(j, carry2):
            pltpu.make_async_copy(
                img_hbm.at[idx_v.at[j]],
                g_v.at[pl.ds(j * 128, 128)],
                sem).start()
            return carry2

        lax.fori_loop(0, K, fire, carry)

        def drain(j, carry2):
            pltpu.make_async_copy(
                img_hbm.at[idx_v.at[j]],
                g_v.at[pl.ds(j * 128, 128)],
                sem).wait()
            return carry2

        lax.fori_loop(0, K, drain, carry)

        pltpu.sync_copy(g_v, out_hbm.at[b, pl.ds(ci * P, P)])
        return carry

    lax.fori_loop(0, NCHUNK, chunk_body, 0)


@jax.jit
def _run(img_aug, tr_rep):
    mesh = plsc.VectorSubcoreMesh(core_axis_name="c", subcore_axis_name="s")
    f = pl.kernel(
        _body,
        mesh=mesh,
        compiler_params=pltpu.CompilerParams(use_tc_tiling_on_sc=False),
        out_type=jax.ShapeDtypeStruct((NB, OH * OW, D), jnp.float32),
        scratch_types=[
            pltpu.VMEM((9, 16), jnp.float32),
            pltpu.VMEM((K, 128), jnp.int32),
            pltpu.VMEM((P, D), jnp.float32),
            pltpu.SemaphoreType.DMA,
        ],
    )
    return f(img_aug, tr_rep)[:, :, :3].reshape(NB, OH, OW, 3)


def kernel(img, transform):
    img_aug = jnp.pad(img.reshape(H * W, 3), ((0, 1), (0, D - 3)))
    tr_rep = jnp.broadcast_to(transform.reshape(NB, 9, 1), (NB, 9, 16))
    return _run(img_aug, tr_rep)
